# Initial kernel scaffold; baseline (speedup 1.0000x reference)
import jax
import jax.numpy as jnp
from jax.experimental import pallas as pl


def kernel(x, z, batch, W, b):
    raise NotImplementedError("write your pallas kernel here")


# speedup vs baseline: 7.7310x; 7.7310x over previous
import functools

import jax
import jax.numpy as jnp
from jax import lax
from jax.experimental import pallas as pl
from jax.experimental.pallas import tpu as pltpu
from jax.experimental.pallas import tpu_sc as plsc

N = 320000
D = 128
S = 10000

NSUB = 16
BLK = 128
ZCHUNK = 16
NCHUNK = S // ZCHUNK
DCHUNK = 200
NDRAIN = S // DCHUNK


def _sc_segment_sums(z, x, idx):
    mesh = plsc.VectorSubcoreMesh(core_axis_name="c", subcore_axis_name="s")

    @functools.partial(
        pl.kernel,
        out_type=[
            jax.ShapeDtypeStruct((S, D), jnp.float32),
            jax.ShapeDtypeStruct((S, D), jnp.float32),
        ],
        mesh=mesh,
        scratch_types=[
            pltpu.VMEM_SHARED((S, D), jnp.float32),
            pltpu.VMEM((ZCHUNK, D), jnp.float32),
            pltpu.VMEM((2, BLK, D), jnp.float32),
            pltpu.VMEM((BLK,), jnp.int32),
            pltpu.VMEM((BLK,), jnp.int32),
            pltpu.SemaphoreType.DMA,
            pltpu.SemaphoreType.DMA,
        ],
    )
    def seg_sums(z_hbm, x_hbm, idx_hbm, zsum_hbm, xsum_hbm, acc, zbuf,
                 rows_v, idx0_v, idx1_v, sem0, sem1):
        cid = lax.axis_index("c")
        sid = lax.axis_index("s")

        @pl.loop(0, ZCHUNK)
        def _(r):
            @pl.loop(0, D, step=16)
            def _(c):
                zbuf.at[pl.ds(r, 1), pl.ds(c, 16)][...] = jnp.zeros(
                    (1, 16), jnp.float32)

        @pl.loop(sid, NCHUNK, step=NSUB)
        def _(c):
            pltpu.async_copy(zbuf, acc.at[pl.ds(c * ZCHUNK, ZCHUNK)], sem0)

        @pl.loop(sid, NCHUNK, step=NSUB)
        def _(c):
            pltpu.make_async_copy(zbuf, acc.at[pl.ds(c * ZCHUNK, ZCHUNK)],
                                  sem0).wait()

        plsc.subcore_barrier()

        nblk = N // BLK
        bps = -(-nblk // NSUB)
        base = sid * bps
        end = jnp.minimum(base + bps, nblk)
        sems = (sem0, sem1)
        idxs = (idx0_v, idx1_v)

        def _start(rows_hbm, j, slot):
            pltpu.async_copy(rows_hbm.at[pl.ds(j * BLK, BLK)],
                             rows_v.at[slot], sems[slot])
            pltpu.async_copy(idx_hbm.at[0, pl.ds(j * BLK, BLK)],
                             idxs[slot], sems[slot])

        def _finish(rows_hbm, j, slot):
            pltpu.make_async_copy(rows_hbm.at[pl.ds(j * BLK, BLK)],
                                  rows_v.at[slot], sems[slot]).wait()
            pltpu.make_async_copy(idx_hbm.at[0, pl.ds(j * BLK, BLK)],
                                  idxs[slot], sems[slot]).wait()
            pltpu.sync_copy(rows_v.at[slot], acc.at[idxs[slot]], add=True)

        def _accumulate(rows_hbm):
            @pl.when(base < end)
            def _():
                _start(rows_hbm, base, 0)

            @pl.loop(0, bps, step=2)
            def _(k):
                j0 = base + k
                j1 = j0 + 1

                @pl.when(j1 < end)
                def _():
                    _start(rows_hbm, j1, 1)

                @pl.when(j0 < end)
                def _():
                    _finish(rows_hbm, j0, 0)

                @pl.when(j1 + 1 < end)
                def _():
                    _start(rows_hbm, j1 + 1, 0)

                @pl.when(j1 < end)
                def _():
                    _finish(rows_hbm, j1, 1)

        @pl.when(cid == 0)
        def _():
            _accumulate(z_hbm)

        @pl.when(cid == 1)
        def _():
            _accumulate(x_hbm)

        plsc.subcore_barrier()

        def _drain(out_hbm):
            @pl.loop(sid, NDRAIN, step=NSUB)
            def _(c):
                pltpu.async_copy(acc.at[pl.ds(c * DCHUNK, DCHUNK)],
                                 out_hbm.at[pl.ds(c * DCHUNK, DCHUNK)], sem0)

            @pl.loop(sid, NDRAIN, step=NSUB)
            def _(c):
                pltpu.make_async_copy(acc.at[pl.ds(c * DCHUNK, DCHUNK)],
                                      out_hbm.at[pl.ds(c * DCHUNK, DCHUNK)],
                                      sem0).wait()

        @pl.when(cid == 0)
        def _():
            _drain(zsum_hbm)

        @pl.when(cid == 1)
        def _():
            _drain(xsum_hbm)

    return seg_sums(z, x, idx)


def _linear_body(zs_ref, xs_ref, wt_ref, b_ref, o_ref):
    wt = wt_ref[...]
    o_ref[...] = (
        jnp.dot(zs_ref[...], wt[:D, :], preferred_element_type=jnp.float32)
        + jnp.dot(xs_ref[...], wt[D:, :], preferred_element_type=jnp.float32)
        + b_ref[...]
    )


def _tc_linear(z_sum, x_sum, wt, b2):
    rows = 1000
    return pl.pallas_call(
        _linear_body,
        grid=(S // rows,),
        in_specs=[
            pl.BlockSpec((rows, D), lambda i: (i, 0)),
            pl.BlockSpec((rows, D), lambda i: (i, 0)),
            pl.BlockSpec((2 * D, D), lambda i: (0, 0)),
            pl.BlockSpec((1, D), lambda i: (0, 0)),
        ],
        out_specs=pl.BlockSpec((rows, D), lambda i: (i, 0)),
        out_shape=jax.ShapeDtypeStruct((S, D), jnp.float32),
    )(z_sum, x_sum, wt, b2)


def kernel(x, z, batch, W, b):
    idx = batch.astype(jnp.int32).reshape(1, N)
    z_sum, x_sum = _sc_segment_sums(z, x, idx)
    return _tc_linear(z_sum, x_sum, W.T, b.reshape(1, D))
